# baseline (speedup 1.0000x reference)
import jax
import jax.numpy as jnp
from jax.experimental import pallas as pl
from jax.experimental.pallas import tpu as pltpu

BSIZE = 32


def _row_kernel(mask_ref, x_ref, w_ref, b_ref, o_ref):
    t = pl.program_id(0)
    rows = x_ref.shape[0]
    w_img = rows // BSIZE
    nbj = w_img // BSIZE

    q = jnp.dot(x_ref[...], w_ref[...], preferred_element_type=jnp.float32)
    q = q + b_ref[...]

    r = jax.lax.broadcasted_iota(jnp.int32, (rows, 1), 0)
    sb = (r % w_img) // BSIZE
    mv = jnp.zeros((rows, 1), jnp.float32)
    for j in range(nbj):
        mv = mv + jnp.where(sb == j, mask_ref[t * nbj + j].astype(jnp.float32), 0.0)
    o_ref[...] = q * mv


def kernel(sbnet_x, active_block_indices, num_active, Wc, bc):
    n_batch, h, w, c = sbnet_x.shape
    oc = Wc.shape[-1]
    nbi = h // BSIZE
    nbj = w // BSIZE
    nblocks = n_batch * nbi * nbj

    idx = active_block_indices
    valid = jnp.arange(idx.shape[0]) < num_active
    flat = (idx[:, 0] * nbi + idx[:, 1]) * nbj + idx[:, 2]
    flat = jnp.where(valid, flat, nblocks)
    mask = jnp.zeros((nblocks,), dtype=jnp.int32).at[flat].set(
        1, mode="drop", unique_indices=True
    )

    x2 = sbnet_x.reshape(n_batch * h * w, c)
    w2 = Wc.reshape(c, oc)
    b2 = bc.reshape(1, oc)
    tile_rows = BSIZE * w

    out = pl.pallas_call(
        _row_kernel,
        grid_spec=pltpu.PrefetchScalarGridSpec(
            num_scalar_prefetch=1,
            grid=(n_batch * nbi,),
            in_specs=[
                pl.BlockSpec((tile_rows, c), lambda t, m: (t, 0)),
                pl.BlockSpec((c, oc), lambda t, m: (0, 0)),
                pl.BlockSpec((1, oc), lambda t, m: (0, 0)),
            ],
            out_specs=pl.BlockSpec((tile_rows, oc), lambda t, m: (t, 0)),
        ),
        out_shape=jax.ShapeDtypeStruct((n_batch * h * w, oc), sbnet_x.dtype),
    )(mask, x2, w2, b2)
    return out.reshape(n_batch, h, w, oc)


# speedup vs baseline: 1.7149x; 1.7149x over previous
import jax
import jax.numpy as jnp
from jax.experimental import pallas as pl
from jax.experimental.pallas import tpu as pltpu

BSIZE = 32


def _row_kernel(mask_ref, x_ref, w_ref, b_ref, o_ref):
    t = pl.program_id(0)
    w_img = o_ref.shape[1]
    nbj = w_img // BSIZE
    c = x_ref.shape[2]

    xb = x_ref[...].reshape(BSIZE * w_img, c)
    q = jnp.dot(xb, w_ref[...], preferred_element_type=jnp.float32)
    q = (q + b_ref[...]).reshape(o_ref.shape)
    for j in range(nbj):
        m = mask_ref[t * nbj + j].astype(jnp.float32)
        lo, hi = j * BSIZE, (j + 1) * BSIZE
        o_ref[:, lo:hi, :] = q[:, lo:hi, :] * m


def kernel(sbnet_x, active_block_indices, num_active, Wc, bc):
    n_batch, h, w, c = sbnet_x.shape
    oc = Wc.shape[-1]
    nbi = h // BSIZE
    nbj = w // BSIZE
    nblocks = n_batch * nbi * nbj

    idx = active_block_indices
    valid = jnp.arange(idx.shape[0]) < num_active
    flat = (idx[:, 0] * nbi + idx[:, 1]) * nbj + idx[:, 2]
    flat = jnp.where(valid, flat, nblocks)
    mask = jnp.zeros((nblocks,), dtype=jnp.int32).at[flat].set(
        1, mode="drop", unique_indices=True
    )

    x3 = sbnet_x.reshape(n_batch * h, w, c)
    w2 = Wc.reshape(c, oc)
    b2 = bc.reshape(1, oc)

    out = pl.pallas_call(
        _row_kernel,
        grid_spec=pltpu.PrefetchScalarGridSpec(
            num_scalar_prefetch=1,
            grid=(n_batch * nbi,),
            in_specs=[
                pl.BlockSpec((BSIZE, w, c), lambda t, m: (t, 0, 0)),
                pl.BlockSpec((c, oc), lambda t, m: (0, 0)),
                pl.BlockSpec((1, oc), lambda t, m: (0, 0)),
            ],
            out_specs=pl.BlockSpec((BSIZE, w, oc), lambda t, m: (t, 0, 0)),
        ),
        out_shape=jax.ShapeDtypeStruct((n_batch * h, w, oc), sbnet_x.dtype),
    )(mask, x3, w2, b2)
    return out.reshape(n_batch, h, w, oc)


# speedup vs baseline: 6.7006x; 3.9072x over previous
import jax
import jax.numpy as jnp
from jax.experimental import pallas as pl
from jax.experimental.pallas import tpu as pltpu

BSIZE = 32


def _row_kernel(mask_ref, x_ref, w_ref, b_ref, o_ref):
    t = pl.program_id(0)
    w_img = x_ref.shape[2]
    nbj = w_img // BSIZE

    lane_blk = jax.lax.broadcasted_iota(jnp.int32, (1, w_img), 1) // BSIZE
    mv = jnp.zeros((1, w_img), jnp.float32)
    for j in range(nbj):
        mv = mv + jnp.where(lane_blk == j,
                            mask_ref[t * nbj + j].astype(jnp.float32), 0.0)

    for r in range(x_ref.shape[0]):
        q = jnp.dot(w_ref[...], x_ref[r], preferred_element_type=jnp.float32)
        o_ref[r] = (q + b_ref[...]) * mv


def kernel(sbnet_x, active_block_indices, num_active, Wc, bc):
    n_batch, h, w, c = sbnet_x.shape
    oc = Wc.shape[-1]
    nbi = h // BSIZE
    nbj = w // BSIZE
    nblocks = n_batch * nbi * nbj

    idx = active_block_indices
    valid = jnp.arange(idx.shape[0]) < num_active
    flat = (idx[:, 0] * nbi + idx[:, 1]) * nbj + idx[:, 2]
    flat = jnp.where(valid, flat, nblocks)
    mask = jnp.zeros((nblocks,), dtype=jnp.int32).at[flat].set(
        1, mode="drop", unique_indices=True
    )

    xt = jnp.transpose(sbnet_x, (0, 1, 3, 2)).reshape(n_batch * h, c, w)
    wt = Wc.reshape(c, oc).T
    b2 = bc.reshape(oc, 1)

    out = pl.pallas_call(
        _row_kernel,
        grid_spec=pltpu.PrefetchScalarGridSpec(
            num_scalar_prefetch=1,
            grid=(n_batch * nbi,),
            in_specs=[
                pl.BlockSpec((BSIZE, c, w), lambda t, m: (t, 0, 0)),
                pl.BlockSpec((oc, c), lambda t, m: (0, 0)),
                pl.BlockSpec((oc, 1), lambda t, m: (0, 0)),
            ],
            out_specs=pl.BlockSpec((BSIZE, oc, w), lambda t, m: (t, 0, 0)),
        ),
        out_shape=jax.ShapeDtypeStruct((n_batch * h, oc, w), sbnet_x.dtype),
    )(mask, xt, wt, b2)
    return out.reshape(n_batch, h, oc, w).transpose(0, 1, 3, 2)


# speedup vs baseline: 6.8216x; 1.0181x over previous
import functools

import jax
import jax.numpy as jnp
from jax.experimental import pallas as pl
from jax.experimental.pallas import tpu as pltpu

BSIZE = 32

_DIMNUMS_CT_LHS = (((0,), (0,)), ((), ()))


def _row_kernel(idx_ref, na_ref, x_ref, w_ref, b_ref, o_ref, *, nbi):
    t = pl.program_id(0)
    w_img = x_ref.shape[2]
    nbj = w_img // BSIZE
    n_idx = idx_ref.shape[0]
    na = na_ref[0]

    def scan_body(k, bits):
        valid = k < na
        rid = idx_ref[k, 0] * nbi + idx_ref[k, 1]
        hit = jnp.logical_and(valid, rid == t)
        return bits | jnp.where(hit, jnp.int32(1) << idx_ref[k, 2], jnp.int32(0))

    bits = jax.lax.fori_loop(0, n_idx, scan_body, jnp.int32(0))

    lane_blk = jax.lax.broadcasted_iota(jnp.int32, (1, w_img), 1) // BSIZE
    mv = jnp.zeros((1, w_img), jnp.float32)
    for j in range(nbj):
        m_j = (bits >> j) & 1
        mv = mv + jnp.where(lane_blk == j, m_j.astype(jnp.float32), 0.0)

    b_col = jnp.transpose(b_ref[...], (1, 0))
    for r in range(x_ref.shape[0]):
        q = jax.lax.dot_general(w_ref[...], x_ref[r], _DIMNUMS_CT_LHS,
                                preferred_element_type=jnp.float32)
        o_ref[r] = (q + b_col) * mv


def kernel(sbnet_x, active_block_indices, num_active, Wc, bc):
    n_batch, h, w, c = sbnet_x.shape
    oc = Wc.shape[-1]
    nbi = h // BSIZE

    na = jnp.reshape(jnp.asarray(num_active, jnp.int32), (1,))

    xt = jnp.transpose(sbnet_x, (0, 1, 3, 2)).reshape(n_batch * h, c, w)
    w2 = Wc.reshape(c, oc)
    b2 = bc.reshape(1, oc)

    out = pl.pallas_call(
        functools.partial(_row_kernel, nbi=nbi),
        grid_spec=pltpu.PrefetchScalarGridSpec(
            num_scalar_prefetch=2,
            grid=(n_batch * nbi,),
            in_specs=[
                pl.BlockSpec((BSIZE, c, w), lambda t, i_, n_: (t, 0, 0)),
                pl.BlockSpec((c, oc), lambda t, i_, n_: (0, 0)),
                pl.BlockSpec((1, oc), lambda t, i_, n_: (0, 0)),
            ],
            out_specs=pl.BlockSpec((BSIZE, oc, w), lambda t, i_, n_: (t, 0, 0)),
        ),
        out_shape=jax.ShapeDtypeStruct((n_batch * h, oc, w), sbnet_x.dtype),
    )(active_block_indices, na, xt, w2, b2)
    return out.reshape(n_batch, h, oc, w).transpose(0, 1, 3, 2)
